# baseline (speedup 1.0000x reference)
import jax
import jax.numpy as jnp
from jax.experimental import pallas as pl
from jax.experimental.pallas import tpu as pltpu

_BLOCK_ROWS = 512


def _add_body(idx_ref, x_ref, emb_ref, o_ref):
    del idx_ref
    o_ref[...] = x_ref[...] + emb_ref[0]


def kernel(x, layer_idx, step_embeddings):
    B, S, D = x.shape
    rows = B * S
    x2 = x.reshape(rows, D)
    emb3 = step_embeddings.reshape(step_embeddings.shape[0], 1, D)
    block = min(_BLOCK_ROWS, rows)
    grid = rows // block
    idx = jnp.asarray(layer_idx, dtype=jnp.int32).reshape(1)
    out = pl.pallas_call(
        _add_body,
        grid_spec=pltpu.PrefetchScalarGridSpec(
            num_scalar_prefetch=1,
            grid=(grid,),
            in_specs=[
                pl.BlockSpec((block, D), lambda i, idx_ref: (i, 0)),
                pl.BlockSpec((1, 1, D), lambda i, idx_ref: (idx_ref[0], 0, 0)),
            ],
            out_specs=pl.BlockSpec((block, D), lambda i, idx_ref: (i, 0)),
        ),
        out_shape=jax.ShapeDtypeStruct((rows, D), x.dtype),
    )(idx, x2, emb3)
    return out.reshape(B, S, D)


# speedup vs baseline: 1.0383x; 1.0383x over previous
import jax
import jax.numpy as jnp
from jax.experimental import pallas as pl
from jax.experimental.pallas import tpu as pltpu

_BLOCK_ROWS = 1024


def _add_body(idx_ref, x_ref, emb_ref, o_ref):
    row = emb_ref[idx_ref[0]]
    o_ref[...] = x_ref[...] + row


def kernel(x, layer_idx, step_embeddings):
    B, S, D = x.shape
    rows = B * S
    x2 = x.reshape(rows, D)
    n_table = step_embeddings.shape[0]
    block = min(_BLOCK_ROWS, rows)
    grid = rows // block
    idx = jnp.asarray(layer_idx, dtype=jnp.int32).reshape(1)
    out = pl.pallas_call(
        _add_body,
        grid=(grid,),
        in_specs=[
            pl.BlockSpec(memory_space=pltpu.SMEM),
            pl.BlockSpec((block, D), lambda i: (i, 0)),
            pl.BlockSpec((n_table, D), lambda i: (0, 0)),
        ],
        out_specs=pl.BlockSpec((block, D), lambda i: (i, 0)),
        out_shape=jax.ShapeDtypeStruct((rows, D), x.dtype),
    )(idx, x2, step_embeddings)
    return out.reshape(B, S, D)


# speedup vs baseline: 1.0400x; 1.0016x over previous
import jax
import jax.numpy as jnp
from jax.experimental import pallas as pl
from jax.experimental.pallas import tpu as pltpu

_BLOCK_ROWS = 1024


def _add_body(idx_ref, x_ref, emb_ref, o_ref):
    row = emb_ref[idx_ref[0]]
    o_ref[...] = x_ref[...] + row


def kernel(x, layer_idx, step_embeddings):
    B, S, D = x.shape
    rows = B * S
    x2 = x.reshape(rows, D)
    n_table = step_embeddings.shape[0]
    block = min(_BLOCK_ROWS, rows)
    grid = rows // block
    idx = jnp.asarray(layer_idx, dtype=jnp.int32).reshape(1)
    out = pl.pallas_call(
        _add_body,
        grid=(grid,),
        in_specs=[
            pl.BlockSpec(memory_space=pltpu.SMEM),
            pl.BlockSpec((block, D), lambda i: (i, 0)),
            pl.BlockSpec((n_table, D), lambda i: (0, 0)),
        ],
        out_specs=pl.BlockSpec((block, D), lambda i: (i, 0)),
        out_shape=jax.ShapeDtypeStruct((rows, D), x.dtype),
        compiler_params=pltpu.CompilerParams(
            dimension_semantics=("parallel",),
        ),
    )(idx, x2, step_embeddings)
    return out.reshape(B, S, D)
